# baseline (speedup 1.0000x reference)
import functools

import jax
import jax.numpy as jnp
from jax import lax
from jax.experimental import pallas as pl
from jax.experimental.pallas import tpu as pltpu
from jax.experimental.pallas import tpu_sc as plsc

N = 10000
E = 160000
D_IN = 128
D_HID = 256
D_LAT = 64

NC, NS = 2, 16
NW = NC * NS
CHUNK = 64
EPW = 5120
E_PAD = NW * EPW
EPC = EPW // CHUNK
N_PAD = 10240
RPT = N_PAD // NS

BM = 1024
GM = 10
DEGW = 128
NBUF = 4
TOTC = E_PAD // CHUNK
CPW0 = 160
CPW1 = 0


def _mesh():
    return plsc.VectorSubcoreMesh(core_axis_name="c", subcore_axis_name="s")


def _deg_body(dst_hbm, zeros_hbm, ones_hbm, out_hbm, idx_v, ones_v, accum):
    c = lax.axis_index("c")
    s = lax.axis_index("s")
    wid = c * NS + s
    pltpu.sync_copy(zeros_hbm, accum.at[pl.ds(s * RPT, RPT)])
    pltpu.sync_copy(ones_hbm, ones_v)
    pltpu.sync_copy(dst_hbm.at[pl.ds(wid * EPC, EPC)], idx_v)
    plsc.subcore_barrier()

    def step(i, carry):
        pltpu.sync_copy(ones_v, accum.at[idx_v.at[i]], add=True)
        return carry

    lax.fori_loop(0, EPC, step, 0)
    plsc.subcore_barrier()
    pltpu.sync_copy(accum.at[pl.ds(s * RPT, RPT)],
                    out_hbm.at[c, pl.ds(s * RPT, RPT)])


def _sc_degree(dst3, zeros_col, ones_col):
    kfn = pl.kernel(
        _deg_body,
        out_type=jax.ShapeDtypeStruct((NC, N_PAD, DEGW), jnp.float32),
        mesh=_mesh(),
        scratch_types=[
            pltpu.VMEM((EPC, CHUNK), jnp.int32),
            pltpu.VMEM((CHUNK, DEGW), jnp.float32),
            pltpu.VMEM_SHARED((N_PAD, DEGW), jnp.float32),
        ],
    )
    return kfn(dst3, zeros_col, ones_col)


def _prop_body(v_hbm, src_hbm, dst_hbm, zeros_hbm, out_hbm,
               sidx, didx, rows2, accum, gsem, isem):
    c = lax.axis_index("c")
    s = lax.axis_index("s")

    start = jnp.where(c == 0, s * CPW0, NS * CPW0 + s * CPW1)
    cnt = jnp.where(c == 0, CPW0, CPW1)

    @pl.when(c == 0)
    def _():
        pltpu.sync_copy(zeros_hbm, accum.at[pl.ds(s * RPT, RPT)])
    plsc.subcore_barrier()

    def fire_idx(i):
        slot = lax.rem(i, NBUF)
        pltpu.async_copy(src_hbm.at[start + i], sidx.at[slot],
                         isem.at[slot, 0])
        pltpu.async_copy(dst_hbm.at[start + i], didx.at[slot],
                         isem.at[slot, 1])

    def fire_gather(i):
        slot = lax.rem(i, NBUF)
        pltpu.make_async_copy(src_hbm.at[start + i], sidx.at[slot],
                              isem.at[slot, 0]).wait()
        pltpu.async_copy(v_hbm.at[sidx.at[slot]], rows2.at[slot],
                         gsem.at[slot])

    for j in range(NBUF - 1):
        @pl.when(j < cnt)
        def _():
            fire_idx(j)
    for j in range(NBUF - 2):
        @pl.when(j < cnt)
        def _():
            fire_gather(j)

    def step(i, carry):
        cur = lax.rem(i, NBUF)
        a2 = i + NBUF - 1
        a1 = i + NBUF - 2

        @pl.when(a2 < cnt)
        def _():
            fire_idx(a2)

        @pl.when(a1 < cnt)
        def _():
            fire_gather(a1)

        pltpu.make_async_copy(v_hbm.at[sidx.at[cur]], rows2.at[cur],
                              gsem.at[cur]).wait()
        pltpu.make_async_copy(dst_hbm.at[start], didx.at[cur],
                              isem.at[cur, 1]).wait()
        pltpu.sync_copy(rows2.at[cur], accum.at[didx.at[cur]], add=True)
        return carry

    lax.fori_loop(0, cnt, step, 0)
    plsc.subcore_barrier()

    @pl.when(c == 0)
    def _():
        pltpu.sync_copy(accum.at[pl.ds(s * RPT, RPT)],
                        out_hbm.at[pl.ds(s * RPT, RPT)])


def _sc_prop(v, src3, dst3, zeros_rows):
    kfn = pl.kernel(
        _prop_body,
        out_type=jax.ShapeDtypeStruct((N_PAD, D_IN), jnp.float32),
        mesh=_mesh(),
        scratch_types=[
            pltpu.VMEM((NBUF, CHUNK), jnp.int32),
            pltpu.VMEM((NBUF, CHUNK), jnp.int32),
            pltpu.VMEM((NBUF, CHUNK, D_IN), jnp.float32),
            pltpu.VMEM_SHARED((N_PAD, D_IN), jnp.float32),
            pltpu.SemaphoreType.DMA((NBUF,)),
            pltpu.SemaphoreType.DMA((NBUF, 2)),
        ],
    )
    return kfn(v, src3, dst3, zeros_rows)


def _prep_tc(degp, x):
    def body(degp_ref, x_ref, dinv_ref, vs1_ref):
        dp = degp_ref[...]
        deg = dp[0, :, 0:1] + dp[1, :, 0:1] + 1.0
        dinv = lax.rsqrt(deg)
        dinv_ref[...] = dinv
        vs1_ref[...] = x_ref[...] * dinv

    return pl.pallas_call(
        body,
        grid=(GM,),
        out_shape=(jax.ShapeDtypeStruct((N, 1), jnp.float32),
                   jax.ShapeDtypeStruct((N, D_IN), jnp.float32)),
        in_specs=[pl.BlockSpec((NC, BM, DEGW), lambda i: (0, i, 0)),
                  pl.BlockSpec((BM, D_IN), lambda i: (i, 0))],
        out_specs=(pl.BlockSpec((BM, 1), lambda i: (i, 0)),
                   pl.BlockSpec((BM, D_IN), lambda i: (i, 0))),
    )(degp, x)


def _encode_tc(p1, vs1, dinv, W1, b1r, Wcat):
    def body(p_ref, vs1_ref, dinv_ref, W1_ref, b1_ref, Wc_ref, out_ref):
        t = dinv_ref[...] * (p_ref[...] + vs1_ref[...])
        h = jnp.dot(t, W1_ref[...], preferred_element_type=jnp.float32)
        h = jnp.maximum(h + b1_ref[...], 0.0)
        hm = jnp.dot(h, Wc_ref[...], preferred_element_type=jnp.float32)
        out_ref[...] = dinv_ref[...] * hm

    return pl.pallas_call(
        body,
        grid=(GM,),
        out_shape=jax.ShapeDtypeStruct((N, D_IN), jnp.float32),
        in_specs=[pl.BlockSpec((BM, D_IN), lambda i: (i, 0)),
                  pl.BlockSpec((BM, D_IN), lambda i: (i, 0)),
                  pl.BlockSpec((BM, 1), lambda i: (i, 0)),
                  pl.BlockSpec((D_IN, D_HID), lambda i: (0, 0)),
                  pl.BlockSpec((1, D_HID), lambda i: (0, 0)),
                  pl.BlockSpec((D_HID, D_IN), lambda i: (0, 0))],
        out_specs=pl.BlockSpec((BM, D_IN), lambda i: (i, 0)),
    )(p1, vs1, dinv, W1, b1r, Wcat)


def _latent_tc(p2, vs2, dinv, bmur, blsr, eps, Wdec, bdecr, mask):
    def body(p_ref, vs2_ref, dinv_ref, bmu_ref, bls_ref, eps_ref,
             Wd_ref, bd_ref, mk_ref, mu_ref, ls_ref, z_ref, xr_ref):
        agg = dinv_ref[...] * (p_ref[...] + vs2_ref[...])
        mu = agg[:, :D_LAT] + bmu_ref[...]
        ls = agg[:, D_LAT:] + bls_ref[...]
        z = mu + eps_ref[...] * jnp.exp(ls)
        mu_ref[...] = mu
        ls_ref[...] = ls
        z_ref[...] = z
        xr_ref[...] = (jnp.dot(z, Wd_ref[...] * mk_ref[...],
                               preferred_element_type=jnp.float32)
                       + bd_ref[...])

    return pl.pallas_call(
        body,
        grid=(GM,),
        out_shape=(jax.ShapeDtypeStruct((N, D_LAT), jnp.float32),
                   jax.ShapeDtypeStruct((N, D_LAT), jnp.float32),
                   jax.ShapeDtypeStruct((N, D_LAT), jnp.float32),
                   jax.ShapeDtypeStruct((N, D_IN), jnp.float32)),
        in_specs=[pl.BlockSpec((BM, D_IN), lambda i: (i, 0)),
                  pl.BlockSpec((BM, D_IN), lambda i: (i, 0)),
                  pl.BlockSpec((BM, 1), lambda i: (i, 0)),
                  pl.BlockSpec((1, D_LAT), lambda i: (0, 0)),
                  pl.BlockSpec((1, D_LAT), lambda i: (0, 0)),
                  pl.BlockSpec((BM, D_LAT), lambda i: (i, 0)),
                  pl.BlockSpec((D_LAT, D_IN), lambda i: (0, 0)),
                  pl.BlockSpec((1, D_IN), lambda i: (0, 0)),
                  pl.BlockSpec((D_LAT, D_IN), lambda i: (0, 0))],
        out_specs=(pl.BlockSpec((BM, D_LAT), lambda i: (i, 0)),
                   pl.BlockSpec((BM, D_LAT), lambda i: (i, 0)),
                   pl.BlockSpec((BM, D_LAT), lambda i: (i, 0)),
                   pl.BlockSpec((BM, D_IN), lambda i: (i, 0))),
    )(p2, vs2, dinv, bmur, blsr, eps, Wdec, bdecr, mask)


def _adj_tc(z):
    def body(zr_ref, zc_ref, out_ref):
        out_ref[...] = lax.dot_general(
            zr_ref[...], zc_ref[...],
            (((1,), (1,)), ((), ())),
            preferred_element_type=jnp.float32)

    return pl.pallas_call(
        body,
        grid=(GM, GM),
        out_shape=jax.ShapeDtypeStruct((N, N), jnp.float32),
        in_specs=[pl.BlockSpec((BM, D_LAT), lambda i, j: (i, 0)),
                  pl.BlockSpec((BM, D_LAT), lambda i, j: (j, 0))],
        out_specs=pl.BlockSpec((BM, BM), lambda i, j: (i, j)),
    )(z, z)


def kernel(x, edge_index, eps, W1, b1, Wmu, bmu, Wls, bls, Wdec, bdec, mask):
    ei = edge_index.astype(jnp.int32)
    pad = E_PAD - E
    src = jnp.concatenate([ei[0], jnp.zeros((pad,), jnp.int32)])
    dst = jnp.concatenate(
        [ei[1], N + (jnp.arange(pad, dtype=jnp.int32) % (N_PAD - N))])
    src3 = src.reshape(TOTC, CHUNK)
    dst3 = dst.reshape(TOTC, CHUNK)
    zeros_col = jnp.zeros((RPT, DEGW), jnp.float32)
    ones_col = jnp.ones((CHUNK, DEGW), jnp.float32)
    zeros_rows = jnp.zeros((RPT, D_IN), jnp.float32)

    degp = _sc_degree(dst3, zeros_col, ones_col)
    dinv, vs1 = _prep_tc(degp, x)
    p1 = _sc_prop(vs1, src3, dst3, zeros_rows)
    vs2 = _encode_tc(p1, vs1, dinv, W1, b1.reshape(1, -1),
                     jnp.concatenate([Wmu, Wls], axis=1))
    p2 = _sc_prop(vs2, src3, dst3, zeros_rows)
    mu, logstd, z, x_recon = _latent_tc(
        p2, vs2, dinv, bmu.reshape(1, -1), bls.reshape(1, -1), eps,
        Wdec, bdec.reshape(1, -1), mask)
    adj = _adj_tc(z)
    return (adj, x_recon, mu, logstd)


# speedup vs baseline: 1.1717x; 1.1717x over previous
import functools

import jax
import jax.numpy as jnp
from jax import lax
from jax.experimental import pallas as pl
from jax.experimental.pallas import tpu as pltpu
from jax.experimental.pallas import tpu_sc as plsc

N = 10000
E = 160000
D_IN = 128
D_HID = 256
D_LAT = 64

NC, NS = 2, 16
NW = NC * NS
CHUNK = 128
EPW = 5120
E_PAD = NW * EPW
EPC = EPW // CHUNK
N_PAD = 10240
RPT = N_PAD // NS

BM = 1024
GM = 10
DEGW = 128
NBUF = 2
TOTC = E_PAD // CHUNK
CPW0 = 65
CPW1 = 15


def _mesh():
    return plsc.VectorSubcoreMesh(core_axis_name="c", subcore_axis_name="s")


def _deg_body(dst_hbm, zeros_hbm, ones_hbm, out_hbm, idx_v, ones_v, accum):
    c = lax.axis_index("c")
    s = lax.axis_index("s")
    wid = c * NS + s
    pltpu.sync_copy(zeros_hbm, accum.at[pl.ds(s * RPT, RPT)])
    pltpu.sync_copy(ones_hbm, ones_v)
    pltpu.sync_copy(dst_hbm.at[pl.ds(wid * EPC, EPC)], idx_v)
    plsc.subcore_barrier()

    def step(i, carry):
        pltpu.sync_copy(ones_v, accum.at[idx_v.at[i]], add=True)
        return carry

    lax.fori_loop(0, EPC, step, 0)
    plsc.subcore_barrier()
    pltpu.sync_copy(accum.at[pl.ds(s * RPT, RPT)],
                    out_hbm.at[c, pl.ds(s * RPT, RPT)])


def _sc_degree(dst3, zeros_col, ones_col):
    kfn = pl.kernel(
        _deg_body,
        out_type=jax.ShapeDtypeStruct((NC, N_PAD, DEGW), jnp.float32),
        mesh=_mesh(),
        scratch_types=[
            pltpu.VMEM((EPC, CHUNK), jnp.int32),
            pltpu.VMEM((CHUNK, DEGW), jnp.float32),
            pltpu.VMEM_SHARED((N_PAD, DEGW), jnp.float32),
        ],
    )
    return kfn(dst3, zeros_col, ones_col)


def _prop_body(v_hbm, src_hbm, dst_hbm, zeros_hbm, out_hbm,
               sidx, didx, rows2, accum, gsem, isem):
    c = lax.axis_index("c")
    s = lax.axis_index("s")

    start = jnp.where(c == 0, s * CPW0, NS * CPW0 + s * CPW1)
    cnt = jnp.where(c == 0, CPW0, CPW1)

    pltpu.sync_copy(zeros_hbm, accum.at[pl.ds(s * RPT, RPT)])
    plsc.subcore_barrier()

    def fire_idx(i):
        slot = lax.rem(i, NBUF)
        pltpu.async_copy(src_hbm.at[start + i], sidx.at[slot],
                         isem.at[slot, 0])
        pltpu.async_copy(dst_hbm.at[start + i], didx.at[slot],
                         isem.at[slot, 1])

    def fire_gather(i):
        slot = lax.rem(i, NBUF)
        pltpu.make_async_copy(src_hbm.at[start + i], sidx.at[slot],
                              isem.at[slot, 0]).wait()
        pltpu.async_copy(v_hbm.at[sidx.at[slot]], rows2.at[slot],
                         gsem.at[slot])

    for j in range(NBUF - 1):
        @pl.when(j < cnt)
        def _():
            fire_idx(j)
    for j in range(NBUF - 2):
        @pl.when(j < cnt)
        def _():
            fire_gather(j)

    def step(i, carry):
        cur = lax.rem(i, NBUF)
        a2 = i + NBUF - 1
        a1 = i + NBUF - 2

        @pl.when(a2 < cnt)
        def _():
            fire_idx(a2)

        @pl.when(a1 < cnt)
        def _():
            fire_gather(a1)

        pltpu.make_async_copy(v_hbm.at[sidx.at[cur]], rows2.at[cur],
                              gsem.at[cur]).wait()
        pltpu.make_async_copy(dst_hbm.at[start], didx.at[cur],
                              isem.at[cur, 1]).wait()
        pltpu.sync_copy(rows2.at[cur], accum.at[didx.at[cur]], add=True)
        return carry

    lax.fori_loop(0, cnt, step, 0)
    plsc.subcore_barrier()
    pltpu.sync_copy(accum.at[pl.ds(s * RPT, RPT)],
                    out_hbm.at[c, pl.ds(s * RPT, RPT)])


def _sc_prop(v, src3, dst3, zeros_rows):
    kfn = pl.kernel(
        _prop_body,
        out_type=jax.ShapeDtypeStruct((NC, N_PAD, D_IN), jnp.float32),
        mesh=_mesh(),
        scratch_types=[
            pltpu.VMEM((NBUF, CHUNK), jnp.int32),
            pltpu.VMEM((NBUF, CHUNK), jnp.int32),
            pltpu.VMEM((NBUF, CHUNK, D_IN), jnp.float32),
            pltpu.VMEM_SHARED((N_PAD, D_IN), jnp.float32),
            pltpu.SemaphoreType.DMA((NBUF,)),
            pltpu.SemaphoreType.DMA((NBUF, 2)),
        ],
    )
    return kfn(v, src3, dst3, zeros_rows)


def _prep_tc(degp, x):
    def body(degp_ref, x_ref, dinv_ref, vs1_ref):
        dp = degp_ref[...]
        deg = dp[0, :, 0:1] + dp[1, :, 0:1] + 1.0
        dinv = lax.rsqrt(deg)
        dinv_ref[...] = dinv
        vs1_ref[...] = x_ref[...] * dinv

    return pl.pallas_call(
        body,
        grid=(GM,),
        out_shape=(jax.ShapeDtypeStruct((N, 1), jnp.float32),
                   jax.ShapeDtypeStruct((N, D_IN), jnp.float32)),
        in_specs=[pl.BlockSpec((NC, BM, DEGW), lambda i: (0, i, 0)),
                  pl.BlockSpec((BM, D_IN), lambda i: (i, 0))],
        out_specs=(pl.BlockSpec((BM, 1), lambda i: (i, 0)),
                   pl.BlockSpec((BM, D_IN), lambda i: (i, 0))),
    )(degp, x)


def _encode_tc(p1, vs1, dinv, W1, b1r, Wcat):
    def body(p_ref, vs1_ref, dinv_ref, W1_ref, b1_ref, Wc_ref, out_ref):
        t = dinv_ref[...] * (p_ref[0] + p_ref[1] + vs1_ref[...])
        h = jnp.dot(t, W1_ref[...], preferred_element_type=jnp.float32)
        h = jnp.maximum(h + b1_ref[...], 0.0)
        hm = jnp.dot(h, Wc_ref[...], preferred_element_type=jnp.float32)
        out_ref[...] = dinv_ref[...] * hm

    return pl.pallas_call(
        body,
        grid=(GM,),
        out_shape=jax.ShapeDtypeStruct((N, D_IN), jnp.float32),
        in_specs=[pl.BlockSpec((NC, BM, D_IN), lambda i: (0, i, 0)),
                  pl.BlockSpec((BM, D_IN), lambda i: (i, 0)),
                  pl.BlockSpec((BM, 1), lambda i: (i, 0)),
                  pl.BlockSpec((D_IN, D_HID), lambda i: (0, 0)),
                  pl.BlockSpec((1, D_HID), lambda i: (0, 0)),
                  pl.BlockSpec((D_HID, D_IN), lambda i: (0, 0))],
        out_specs=pl.BlockSpec((BM, D_IN), lambda i: (i, 0)),
    )(p1, vs1, dinv, W1, b1r, Wcat)


def _latent_tc(p2, vs2, dinv, bmur, blsr, eps, Wdec, bdecr, mask):
    def body(p_ref, vs2_ref, dinv_ref, bmu_ref, bls_ref, eps_ref,
             Wd_ref, bd_ref, mk_ref, mu_ref, ls_ref, z_ref, xr_ref):
        agg = dinv_ref[...] * (p_ref[0] + p_ref[1] + vs2_ref[...])
        mu = agg[:, :D_LAT] + bmu_ref[...]
        ls = agg[:, D_LAT:] + bls_ref[...]
        z = mu + eps_ref[...] * jnp.exp(ls)
        mu_ref[...] = mu
        ls_ref[...] = ls
        z_ref[...] = z
        xr_ref[...] = (jnp.dot(z, Wd_ref[...] * mk_ref[...],
                               preferred_element_type=jnp.float32)
                       + bd_ref[...])

    return pl.pallas_call(
        body,
        grid=(GM,),
        out_shape=(jax.ShapeDtypeStruct((N, D_LAT), jnp.float32),
                   jax.ShapeDtypeStruct((N, D_LAT), jnp.float32),
                   jax.ShapeDtypeStruct((N, D_LAT), jnp.float32),
                   jax.ShapeDtypeStruct((N, D_IN), jnp.float32)),
        in_specs=[pl.BlockSpec((NC, BM, D_IN), lambda i: (0, i, 0)),
                  pl.BlockSpec((BM, D_IN), lambda i: (i, 0)),
                  pl.BlockSpec((BM, 1), lambda i: (i, 0)),
                  pl.BlockSpec((1, D_LAT), lambda i: (0, 0)),
                  pl.BlockSpec((1, D_LAT), lambda i: (0, 0)),
                  pl.BlockSpec((BM, D_LAT), lambda i: (i, 0)),
                  pl.BlockSpec((D_LAT, D_IN), lambda i: (0, 0)),
                  pl.BlockSpec((1, D_IN), lambda i: (0, 0)),
                  pl.BlockSpec((D_LAT, D_IN), lambda i: (0, 0))],
        out_specs=(pl.BlockSpec((BM, D_LAT), lambda i: (i, 0)),
                   pl.BlockSpec((BM, D_LAT), lambda i: (i, 0)),
                   pl.BlockSpec((BM, D_LAT), lambda i: (i, 0)),
                   pl.BlockSpec((BM, D_IN), lambda i: (i, 0))),
    )(p2, vs2, dinv, bmur, blsr, eps, Wdec, bdecr, mask)


def _adj_tc(z):
    def body(zr_ref, zc_ref, out_ref):
        out_ref[...] = lax.dot_general(
            zr_ref[...], zc_ref[...],
            (((1,), (1,)), ((), ())),
            preferred_element_type=jnp.float32)

    return pl.pallas_call(
        body,
        grid=(GM, GM),
        out_shape=jax.ShapeDtypeStruct((N, N), jnp.float32),
        in_specs=[pl.BlockSpec((BM, D_LAT), lambda i, j: (i, 0)),
                  pl.BlockSpec((BM, D_LAT), lambda i, j: (j, 0))],
        out_specs=pl.BlockSpec((BM, BM), lambda i, j: (i, j)),
    )(z, z)


def kernel(x, edge_index, eps, W1, b1, Wmu, bmu, Wls, bls, Wdec, bdec, mask):
    ei = edge_index.astype(jnp.int32)
    pad = E_PAD - E
    src = jnp.concatenate([ei[0], jnp.zeros((pad,), jnp.int32)])
    dst = jnp.concatenate(
        [ei[1], N + (jnp.arange(pad, dtype=jnp.int32) % (N_PAD - N))])
    src3 = src.reshape(TOTC, CHUNK)
    dst3 = dst.reshape(TOTC, CHUNK)
    zeros_col = jnp.zeros((RPT, DEGW), jnp.float32)
    ones_col = jnp.ones((CHUNK, DEGW), jnp.float32)
    zeros_rows = jnp.zeros((RPT, D_IN), jnp.float32)

    degp = _sc_degree(dst3, zeros_col, ones_col)
    dinv, vs1 = _prep_tc(degp, x)
    p1 = _sc_prop(vs1, src3, dst3, zeros_rows)
    vs2 = _encode_tc(p1, vs1, dinv, W1, b1.reshape(1, -1),
                     jnp.concatenate([Wmu, Wls], axis=1))
    p2 = _sc_prop(vs2, src3, dst3, zeros_rows)
    mu, logstd, z, x_recon = _latent_tc(
        p2, vs2, dinv, bmu.reshape(1, -1), bls.reshape(1, -1), eps,
        Wdec, bdec.reshape(1, -1), mask)
    adj = _adj_tc(z)
    return (adj, x_recon, mu, logstd)


# speedup vs baseline: 1.1718x; 1.0001x over previous
import functools

import jax
import jax.numpy as jnp
from jax import lax
from jax.experimental import pallas as pl
from jax.experimental.pallas import tpu as pltpu
from jax.experimental.pallas import tpu_sc as plsc

N = 10000
E = 160000
D_IN = 128
D_HID = 256
D_LAT = 64

NC, NS = 2, 16
NW = NC * NS
CHUNK = 128
EPW = 5120
E_PAD = NW * EPW
EPC = EPW // CHUNK
N_PAD = 10240
RPT = N_PAD // NS

BM = 1024
GM = 10
DEGW = 128
NBUF = 2
TOTC = E_PAD // CHUNK
CPW0 = 65
CPW1 = 15


def _mesh():
    return plsc.VectorSubcoreMesh(core_axis_name="c", subcore_axis_name="s")


def _deg_body(dst_hbm, zeros_hbm, ones_hbm, out_hbm, idx_v, ones_v, accum):
    c = lax.axis_index("c")
    s = lax.axis_index("s")
    wid = c * NS + s
    pltpu.sync_copy(zeros_hbm, accum.at[pl.ds(s * RPT, RPT)])
    pltpu.sync_copy(ones_hbm, ones_v)
    pltpu.sync_copy(dst_hbm.at[pl.ds(wid * EPC, EPC)], idx_v)
    plsc.subcore_barrier()

    def step(i, carry):
        pltpu.sync_copy(ones_v, accum.at[idx_v.at[i]], add=True)
        return carry

    lax.fori_loop(0, EPC, step, 0)
    plsc.subcore_barrier()
    pltpu.sync_copy(accum.at[pl.ds(s * RPT, RPT)],
                    out_hbm.at[c, pl.ds(s * RPT, RPT)])


def _sc_degree(dst3, zeros_col, ones_col):
    kfn = pl.kernel(
        _deg_body,
        out_type=jax.ShapeDtypeStruct((NC, N_PAD, DEGW), jnp.float32),
        mesh=_mesh(),
        scratch_types=[
            pltpu.VMEM((EPC, CHUNK), jnp.int32),
            pltpu.VMEM((CHUNK, DEGW), jnp.float32),
            pltpu.VMEM_SHARED((N_PAD, DEGW), jnp.float32),
        ],
    )
    return kfn(dst3, zeros_col, ones_col)


def _prop_body(v_hbm, src_hbm, dst_hbm, zeros_hbm, out_hbm,
               sidx, didx, rows2, accum, gsem, isem):
    c = lax.axis_index("c")
    s = lax.axis_index("s")

    start = jnp.where(c == 0, s * CPW0, NS * CPW0 + s * CPW1)
    cnt = jnp.where(c == 0, CPW0, CPW1)

    pltpu.sync_copy(zeros_hbm, accum.at[pl.ds(s * RPT, RPT)])
    plsc.subcore_barrier()

    def fire_idx(i):
        slot = lax.rem(i, NBUF)
        pltpu.async_copy(src_hbm.at[start + i], sidx.at[slot],
                         isem.at[slot, 0])
        pltpu.async_copy(dst_hbm.at[start + i], didx.at[slot],
                         isem.at[slot, 1])

    def fire_gather(i):
        slot = lax.rem(i, NBUF)
        pltpu.make_async_copy(src_hbm.at[start + i], sidx.at[slot],
                              isem.at[slot, 0]).wait()
        pltpu.async_copy(v_hbm.at[sidx.at[slot]], rows2.at[slot],
                         gsem.at[slot])

    for j in range(NBUF - 1):
        @pl.when(j < cnt)
        def _():
            fire_idx(j)
    for j in range(NBUF - 2):
        @pl.when(j < cnt)
        def _():
            fire_gather(j)

    def step(i, carry):
        cur = lax.rem(i, NBUF)
        a2 = i + NBUF - 1
        a1 = i + NBUF - 2

        @pl.when(a2 < cnt)
        def _():
            fire_idx(a2)

        @pl.when(a1 < cnt)
        def _():
            fire_gather(a1)

        pltpu.make_async_copy(v_hbm.at[sidx.at[cur]], rows2.at[cur],
                              gsem.at[cur]).wait()
        pltpu.make_async_copy(dst_hbm.at[start], didx.at[cur],
                              isem.at[cur, 1]).wait()
        pltpu.sync_copy(rows2.at[cur], accum.at[didx.at[cur]], add=True)
        return carry

    lax.fori_loop(0, cnt, step, 0)
    plsc.subcore_barrier()
    pltpu.sync_copy(accum.at[pl.ds(s * RPT, RPT)],
                    out_hbm.at[c, pl.ds(s * RPT, RPT)])


def _sc_prop(v, src3, dst3, zeros_rows):
    kfn = pl.kernel(
        _prop_body,
        out_type=jax.ShapeDtypeStruct((NC, N_PAD, D_IN), jnp.float32),
        mesh=_mesh(),
        scratch_types=[
            pltpu.VMEM((NBUF, CHUNK), jnp.int32),
            pltpu.VMEM((NBUF, CHUNK), jnp.int32),
            pltpu.VMEM((NBUF, CHUNK, D_IN), jnp.float32),
            pltpu.VMEM_SHARED((N_PAD, D_IN), jnp.float32),
            pltpu.SemaphoreType.DMA((NBUF,)),
            pltpu.SemaphoreType.DMA((NBUF, 2)),
        ],
    )
    return kfn(v, src3, dst3, zeros_rows)


def _prep_tc(degp, x):
    def body(degp_ref, x_ref, dinv_ref, vs1_ref):
        dp = degp_ref[...]
        deg = dp[0, :, 0:1] + dp[1, :, 0:1] + 1.0
        dinv = lax.rsqrt(deg)
        dinv_ref[...] = dinv
        vs1_ref[...] = x_ref[...] * dinv

    return pl.pallas_call(
        body,
        grid=(GM,),
        out_shape=(jax.ShapeDtypeStruct((N, 1), jnp.float32),
                   jax.ShapeDtypeStruct((N, D_IN), jnp.float32)),
        in_specs=[pl.BlockSpec((NC, BM, DEGW), lambda i: (0, i, 0)),
                  pl.BlockSpec((BM, D_IN), lambda i: (i, 0))],
        out_specs=(pl.BlockSpec((BM, 1), lambda i: (i, 0)),
                   pl.BlockSpec((BM, D_IN), lambda i: (i, 0))),
    )(degp, x)


def _encode_tc(p1, vs1, dinv, W1, b1r, Wcat):
    def body(p_ref, vs1_ref, dinv_ref, W1_ref, b1_ref, Wc_ref, out_ref):
        t = dinv_ref[...] * (p_ref[0] + p_ref[1] + vs1_ref[...])
        h = jnp.dot(t, W1_ref[...], preferred_element_type=jnp.float32)
        h = jnp.maximum(h + b1_ref[...], 0.0)
        hm = jnp.dot(h, Wc_ref[...], preferred_element_type=jnp.float32)
        out_ref[...] = dinv_ref[...] * hm

    return pl.pallas_call(
        body,
        grid=(GM,),
        out_shape=jax.ShapeDtypeStruct((N, D_IN), jnp.float32),
        in_specs=[pl.BlockSpec((NC, BM, D_IN), lambda i: (0, i, 0)),
                  pl.BlockSpec((BM, D_IN), lambda i: (i, 0)),
                  pl.BlockSpec((BM, 1), lambda i: (i, 0)),
                  pl.BlockSpec((D_IN, D_HID), lambda i: (0, 0)),
                  pl.BlockSpec((1, D_HID), lambda i: (0, 0)),
                  pl.BlockSpec((D_HID, D_IN), lambda i: (0, 0))],
        out_specs=pl.BlockSpec((BM, D_IN), lambda i: (i, 0)),
    )(p1, vs1, dinv, W1, b1r, Wcat)


def _latent_tc(p2, vs2, dinv, bmur, blsr, eps, Wdec, bdecr, mask):
    def body(p_ref, vs2_ref, dinv_ref, bmu_ref, bls_ref, eps_ref,
             Wd_ref, bd_ref, mk_ref, mu_ref, ls_ref, z_ref, xr_ref):
        agg = dinv_ref[...] * (p_ref[0] + p_ref[1] + vs2_ref[...])
        mu = agg[:, :D_LAT] + bmu_ref[...]
        ls = agg[:, D_LAT:] + bls_ref[...]
        z = mu + eps_ref[...] * jnp.exp(ls)
        mu_ref[...] = mu
        ls_ref[...] = ls
        z_ref[...] = z
        xr_ref[...] = (jnp.dot(z, Wd_ref[...] * mk_ref[...],
                               preferred_element_type=jnp.float32)
                       + bd_ref[...])

    return pl.pallas_call(
        body,
        grid=(GM,),
        out_shape=(jax.ShapeDtypeStruct((N, D_LAT), jnp.float32),
                   jax.ShapeDtypeStruct((N, D_LAT), jnp.float32),
                   jax.ShapeDtypeStruct((N, D_LAT), jnp.float32),
                   jax.ShapeDtypeStruct((N, D_IN), jnp.float32)),
        in_specs=[pl.BlockSpec((NC, BM, D_IN), lambda i: (0, i, 0)),
                  pl.BlockSpec((BM, D_IN), lambda i: (i, 0)),
                  pl.BlockSpec((BM, 1), lambda i: (i, 0)),
                  pl.BlockSpec((1, D_LAT), lambda i: (0, 0)),
                  pl.BlockSpec((1, D_LAT), lambda i: (0, 0)),
                  pl.BlockSpec((BM, D_LAT), lambda i: (i, 0)),
                  pl.BlockSpec((D_LAT, D_IN), lambda i: (0, 0)),
                  pl.BlockSpec((1, D_IN), lambda i: (0, 0)),
                  pl.BlockSpec((D_LAT, D_IN), lambda i: (0, 0))],
        out_specs=(pl.BlockSpec((BM, D_LAT), lambda i: (i, 0)),
                   pl.BlockSpec((BM, D_LAT), lambda i: (i, 0)),
                   pl.BlockSpec((BM, D_LAT), lambda i: (i, 0)),
                   pl.BlockSpec((BM, D_IN), lambda i: (i, 0))),
    )(p2, vs2, dinv, bmur, blsr, eps, Wdec, bdecr, mask)


def _adj_tc(z):
    def body(zr_ref, zc_ref, out_ref):
        out_ref[...] = lax.dot_general(
            zr_ref[...].astype(jnp.bfloat16), zc_ref[...].astype(jnp.bfloat16),
            (((1,), (1,)), ((), ())),
            preferred_element_type=jnp.float32)

    return pl.pallas_call(
        body,
        grid=(GM, GM),
        out_shape=jax.ShapeDtypeStruct((N, N), jnp.float32),
        in_specs=[pl.BlockSpec((BM, D_LAT), lambda i, j: (i, 0)),
                  pl.BlockSpec((BM, D_LAT), lambda i, j: (j, 0))],
        out_specs=pl.BlockSpec((BM, BM), lambda i, j: (i, j)),
    )(z, z)


def kernel(x, edge_index, eps, W1, b1, Wmu, bmu, Wls, bls, Wdec, bdec, mask):
    ei = edge_index.astype(jnp.int32)
    pad = E_PAD - E
    src = jnp.concatenate([ei[0], jnp.zeros((pad,), jnp.int32)])
    dst = jnp.concatenate(
        [ei[1], N + (jnp.arange(pad, dtype=jnp.int32) % (N_PAD - N))])
    src3 = src.reshape(TOTC, CHUNK)
    dst3 = dst.reshape(TOTC, CHUNK)
    zeros_col = jnp.zeros((RPT, DEGW), jnp.float32)
    ones_col = jnp.ones((CHUNK, DEGW), jnp.float32)
    zeros_rows = jnp.zeros((RPT, D_IN), jnp.float32)

    degp = _sc_degree(dst3, zeros_col, ones_col)
    dinv, vs1 = _prep_tc(degp, x)
    p1 = _sc_prop(vs1, src3, dst3, zeros_rows)
    vs2 = _encode_tc(p1, vs1, dinv, W1, b1.reshape(1, -1),
                     jnp.concatenate([Wmu, Wls], axis=1))
    p2 = _sc_prop(vs2, src3, dst3, zeros_rows)
    mu, logstd, z, x_recon = _latent_tc(
        p2, vs2, dinv, bmu.reshape(1, -1), bls.reshape(1, -1), eps,
        Wdec, bdec.reshape(1, -1), mask)
    adj = _adj_tc(z)
    return (adj, x_recon, mu, logstd)
